# baseline (speedup 1.0000x reference)
import functools
import math

import jax
import jax.numpy as jnp
from jax.experimental import pallas as pl
from jax.experimental.pallas import tpu as pltpu

_LN2 = math.log(2.0)
_LOG2E = 1.0 / _LN2


def _loss_kernel(learn_tile_ref, fix_tile_ref, out_ref,
                 colsum_ref, rowsum_ref, rowacc_ref,
                 *, nr, nc, tm, inv_temp, half_weight):
    r = pl.program_id(0)
    c = pl.program_id(1)
    n = nr * tm
    first = jnp.logical_and(r == 0, c == 0)

    learn_t32 = learn_tile_ref[...] * (inv_temp * _LOG2E)
    learn_bf = learn_t32.astype(jnp.bfloat16)
    fb = fix_tile_ref[...].astype(jnp.bfloat16)

    dn = (((1,), (1,)), ((), ()))
    nk = 4
    hh = tm // nk
    cparts, rpart = [], None
    for k in range(nk):
        fbk = fb[k * hh:(k + 1) * hh, :]
        sim = jax.lax.dot_general(learn_bf, fbk, dn,
                                  preferred_element_type=jnp.float32)
        e = jnp.exp2(sim)
        cparts.append(jnp.sum(e.reshape(tm // 8, 8, hh), axis=0))
        rk = jnp.sum(e, axis=1, keepdims=True)
        rpart = rk if rpart is None else rpart + rk

    cpart = jnp.concatenate(cparts, axis=1)
    prev_c = colsum_ref[:, pl.ds(c * tm, tm)]
    colsum_ref[:, pl.ds(c * tm, tm)] = jnp.where(r == 0, cpart, prev_c + cpart)

    prev_r = rowsum_ref[...]
    row_sum = jnp.where(c == 0, rpart, prev_r + rpart)
    rowsum_ref[...] = row_sum

    diag2 = jnp.sum(learn_t32 * fb.astype(jnp.float32),
                    axis=1, keepdims=True)
    dterm = jnp.where(c == r, jnp.sum(diag2), 0.0)
    lterm = jnp.where(c == nc - 1,
                      half_weight * jnp.sum(jnp.log2(row_sum)), 0.0)
    prev_acc = jnp.where(first, 0.0, rowacc_ref[0, 0])
    total = prev_acc + lterm - dterm
    rowacc_ref[0, 0] = total

    col_total = half_weight * jnp.sum(
        jnp.log2(jnp.sum(colsum_ref[...], axis=0, keepdims=True)))
    out_ref[0, 0] = _LN2 * (total + col_total) / n


def _pick_row_tile(n):
    for t in (1024, 512, 256, 128, 64, 32, 16, 8):
        if n % t == 0:
            return t
    return n


def _t2t_loss(learn, fix, *, temperature=0.07, loss_weight=1.0, tm=None):
    assert learn.ndim == 2 and learn.shape == fix.shape
    n, d = learn.shape

    if tm is None:
        tm = _pick_row_tile(n)
    nr = nc = n // tm

    body = functools.partial(
        _loss_kernel,
        nr=nr, nc=nc, tm=tm,
        inv_temp=1.0 / temperature,
        half_weight=0.5 * float(loss_weight),
    )

    out = pl.pallas_call(
        body,
        out_shape=jax.ShapeDtypeStruct((1, 1), jnp.float32),
        grid=(nr, nc),
        in_specs=[
            pl.BlockSpec((tm, d), lambda r, c: (r, 0)),
            pl.BlockSpec((tm, d), lambda r, c: (c, 0)),
        ],
        out_specs=pl.BlockSpec(memory_space=pltpu.SMEM),
        scratch_shapes=[
            pltpu.VMEM((8, n), jnp.float32),
            pltpu.VMEM((tm, 1), jnp.float32),
            pltpu.SMEM((1, 1), jnp.float32),
        ],
        compiler_params=pltpu.CompilerParams(
            dimension_semantics=("arbitrary", "arbitrary"),
            vmem_limit_bytes=64 * 2 ** 20),
    )(learn.astype(jnp.float32), fix.astype(jnp.float32))

    return jnp.reshape(out, ())


def kernel(learn, fix):
    return _t2t_loss(learn, fix, temperature=0.07, loss_weight=1.0)


# speedup vs baseline: 1.0191x; 1.0191x over previous
import functools
import math

import jax
import jax.numpy as jnp
from jax.experimental import pallas as pl
from jax.experimental.pallas import tpu as pltpu

_LN2 = math.log(2.0)
_LOG2E = 1.0 / _LN2


def _loss_kernel(learn_tile_ref, fix_tile_ref, out_ref,
                 colsum_ref, rowsum_ref, rowacc_ref,
                 *, nr, nc, tm, inv_temp, half_weight):
    r = pl.program_id(0)
    c = pl.program_id(1)
    n = nr * tm
    first = jnp.logical_and(r == 0, c == 0)

    learn_t32 = learn_tile_ref[...] * (inv_temp * _LOG2E)
    learn_bf = learn_t32.astype(jnp.bfloat16)
    fb = fix_tile_ref[...].astype(jnp.bfloat16)

    dn = (((1,), (1,)), ((), ()))
    sim = jax.lax.dot_general(learn_bf, fb, dn,
                              preferred_element_type=jnp.float32)
    e = jnp.exp2(sim)

    cpart = jnp.sum(e.reshape(tm // 8, 8, tm), axis=0)
    prev_c = colsum_ref[:, pl.ds(c * tm, tm)]
    colsum_ref[:, pl.ds(c * tm, tm)] = jnp.where(r == 0, cpart, prev_c + cpart)

    rpart = jnp.sum(e, axis=1, keepdims=True)
    prev_r = rowsum_ref[...]
    row_sum = jnp.where(c == 0, rpart, prev_r + rpart)
    rowsum_ref[...] = row_sum

    diag2 = jnp.sum(learn_t32 * fb.astype(jnp.float32),
                    axis=1, keepdims=True)
    dterm = jnp.where(c == r, jnp.sum(diag2), 0.0)
    lterm = jnp.where(c == nc - 1,
                      half_weight * jnp.sum(jnp.log2(row_sum)), 0.0)
    prev_acc = jnp.where(first, 0.0, rowacc_ref[0, 0])
    total = prev_acc + lterm - dterm
    rowacc_ref[0, 0] = total

    col_total = half_weight * jnp.sum(
        jnp.log2(jnp.sum(colsum_ref[...], axis=0, keepdims=True)))
    out_ref[0, 0] = _LN2 * (total + col_total) / n


def _pick_row_tile(n):
    for t in (1024, 512, 256, 128, 64, 32, 16, 8):
        if n % t == 0:
            return t
    return n


def _t2t_loss(learn, fix, *, temperature=0.07, loss_weight=1.0, tm=None):
    assert learn.ndim == 2 and learn.shape == fix.shape
    n, d = learn.shape

    if tm is None:
        tm = _pick_row_tile(n)
    nr = nc = n // tm

    body = functools.partial(
        _loss_kernel,
        nr=nr, nc=nc, tm=tm,
        inv_temp=1.0 / temperature,
        half_weight=0.5 * float(loss_weight),
    )

    out = pl.pallas_call(
        body,
        out_shape=jax.ShapeDtypeStruct((1, 1), jnp.float32),
        grid=(nr, nc),
        in_specs=[
            pl.BlockSpec((tm, d), lambda r, c: (r, 0)),
            pl.BlockSpec((tm, d), lambda r, c: (c, 0)),
        ],
        out_specs=pl.BlockSpec(memory_space=pltpu.SMEM),
        scratch_shapes=[
            pltpu.VMEM((8, n), jnp.float32),
            pltpu.VMEM((tm, 1), jnp.float32),
            pltpu.SMEM((1, 1), jnp.float32),
        ],
        compiler_params=pltpu.CompilerParams(
            dimension_semantics=("arbitrary", "arbitrary"),
            vmem_limit_bytes=64 * 2 ** 20),
    )(learn.astype(jnp.float32), fix.astype(jnp.float32))

    return jnp.reshape(out, ())


def kernel(learn, fix):
    return _t2t_loss(learn, fix, temperature=0.07, loss_weight=1.0)
